# baseline (speedup 1.0000x reference)
import jax
import jax.numpy as jnp
from jax.experimental import pallas as pl
from jax.experimental.pallas import tpu as pltpu

N = 10000
F_IN = 128
F_OUT = 128
BM = 400
NSTEPS = N // BM
NBUF = 3


def _gcn_kernel(x_ref, w_ref, adj_hbm, out_hbm, xw_ref, bufs, stage,
                sems, osems):
    for b in range(NBUF):
        pltpu.make_async_copy(
            adj_hbm.at[pl.ds(b * BM, BM), :], bufs.at[b], sems.at[b]
        ).start()

    xw_ref[...] = jnp.dot(x_ref[...], w_ref[...],
                          preferred_element_type=jnp.float32)

    def body(i, carry):
        slot = jax.lax.rem(i, NBUF)
        oslot = jax.lax.rem(i, 2)
        pltpu.make_async_copy(
            adj_hbm.at[pl.ds(i * BM, BM), :], bufs.at[slot], sems.at[slot]
        ).wait()

        @pl.when(i >= 2)
        def _():
            pltpu.make_async_copy(
                stage.at[oslot], out_hbm.at[pl.ds((i - 2) * BM, BM), :],
                osems.at[oslot]
            ).wait()

        acc = jnp.dot(bufs[slot], xw_ref[...],
                      preferred_element_type=jnp.float32)
        stage[oslot] = jnp.maximum(acc, 0.0)
        pltpu.make_async_copy(
            stage.at[oslot], out_hbm.at[pl.ds(i * BM, BM), :],
            osems.at[oslot]
        ).start()

        @pl.when(i + NBUF < NSTEPS)
        def _():
            pltpu.make_async_copy(
                adj_hbm.at[pl.ds((i + NBUF) * BM, BM), :],
                bufs.at[slot], sems.at[slot]
            ).start()

        return carry

    jax.lax.fori_loop(0, NSTEPS, body, 0)

    for j in (NSTEPS - 2, NSTEPS - 1):
        pltpu.make_async_copy(
            stage.at[j % 2], out_hbm.at[pl.ds(j * BM, BM), :],
            osems.at[j % 2]
        ).wait()


def kernel(input, adj, weight):
    return pl.pallas_call(
        _gcn_kernel,
        in_specs=[
            pl.BlockSpec(memory_space=pltpu.VMEM),
            pl.BlockSpec(memory_space=pltpu.VMEM),
            pl.BlockSpec(memory_space=pltpu.MemorySpace.HBM),
        ],
        out_specs=pl.BlockSpec(memory_space=pltpu.MemorySpace.HBM),
        out_shape=jax.ShapeDtypeStruct((N, F_OUT), jnp.float32),
        scratch_shapes=[
            pltpu.VMEM((N, F_OUT), jnp.float32),
            pltpu.VMEM((NBUF, BM, N), jnp.float32),
            pltpu.VMEM((2, BM, F_OUT), jnp.float32),
            pltpu.SemaphoreType.DMA((NBUF,)),
            pltpu.SemaphoreType.DMA((2,)),
        ],
    )(input, weight, adj)


# speedup vs baseline: 1.0387x; 1.0387x over previous
import jax
import jax.numpy as jnp
from jax.experimental import pallas as pl
from jax.experimental.pallas import tpu as pltpu

N = 10000
F_IN = 128
F_OUT = 128
BM = 400


def _gcn_kernel(x_ref, w_ref, adj_ref, out_ref, xw_ref):
    @pl.when(pl.program_id(0) == 0)
    def _():
        xw_ref[...] = jnp.dot(x_ref[...], w_ref[...],
                              preferred_element_type=jnp.float32)

    acc = jnp.dot(adj_ref[...], xw_ref[...],
                  preferred_element_type=jnp.float32)
    out_ref[...] = jnp.maximum(acc, 0.0)


def kernel(input, adj, weight):
    grid = (N // BM,)
    return pl.pallas_call(
        _gcn_kernel,
        grid=grid,
        in_specs=[
            pl.BlockSpec((N, F_IN), lambda i: (0, 0)),
            pl.BlockSpec((F_IN, F_OUT), lambda i: (0, 0)),
            pl.BlockSpec((BM, N), lambda i: (i, 0)),
        ],
        out_specs=pl.BlockSpec((BM, F_OUT), lambda i: (i, 0)),
        out_shape=jax.ShapeDtypeStruct((N, F_OUT), jnp.float32),
        scratch_shapes=[pltpu.VMEM((N, F_OUT), jnp.float32)],
        compiler_params=pltpu.CompilerParams(
            dimension_semantics=("arbitrary",),
        ),
    )(input, weight, adj)
